# Initial kernel scaffold; baseline (speedup 1.0000x reference)
import jax
import jax.numpy as jnp
from jax.experimental import pallas as pl


def kernel(x, table):
    raise NotImplementedError("write your pallas kernel here")


# speedup vs baseline: 2.4980x; 2.4980x over previous
import functools

import jax
import jax.numpy as jnp
from jax import lax
from jax.experimental import pallas as pl
from jax.experimental.pallas import tpu as pltpu
from jax.experimental.pallas import tpu_sc as plsc

B = 16384
S = 200
D = 64
NC = 2
NS = 16
NW = NC * NS
BPW = B // NW
CB = 4
NCH = BPW // CB
G = 80
NG = (CB * S) // G
L = 16
NVR = D // L

_mesh = plsc.VectorSubcoreMesh(core_axis_name="c", subcore_axis_name="s")


@functools.partial(
    pl.kernel,
    mesh=_mesh,
    out_type=jax.ShapeDtypeStruct((B, D), jnp.float32),
    scratch_types=[
        pltpu.VMEM((CB * S,), jnp.int32),
        pltpu.VMEM((CB * S, D), jnp.float32),
        pltpu.VMEM((CB, D), jnp.float32),
        pltpu.VMEM((L,), jnp.float32),
        pltpu.SemaphoreType.DMA,
    ],
    compiler_params=pltpu.CompilerParams(use_tc_tiling_on_sc=False),
)
def _emb_pool(x_hbm, table_hbm, out_hbm, idx_v, rows_v, out_v, cnt_v, gat_sem):
    wid = lax.axis_index("s") * NC + lax.axis_index("c")
    lane = lax.broadcasted_iota(jnp.int32, (L,), 0)
    lo8 = jnp.minimum(jnp.maximum(8 - lane, 0), 1)
    hi8 = 1 - lo8

    def chunk_body(i, carry):
        row0 = wid * BPW + i * CB
        pltpu.sync_copy(x_hbm.at[pl.ds(row0 * S, CB * S)], idx_v)
        for j in range(NG):
            pltpu.async_copy(
                table_hbm.at[idx_v.at[pl.ds(j * G, G)]],
                rows_v.at[pl.ds(j * G, G)],
                gat_sem,
            )
        for j in range(NG):
            pltpu.make_async_copy(
                table_hbm.at[idx_v.at[pl.ds(j * G, G)]],
                rows_v.at[pl.ds(j * G, G)],
                gat_sem,
            ).wait()

        for r in range(CB):
            def srow(s, accs):
                row = r * S + s
                return tuple(
                    a + rows_v[row, pl.ds(L * l, L)] for l, a in enumerate(accs)
                )

            accs = lax.fori_loop(
                0, S, srow, tuple(jnp.zeros((L,), jnp.float32) for _ in range(NVR))
            )

            if r % 2 == 0:
                full0 = (S * r) // L
                shared_k = full0 + 12
                shared_mask = lo8
            else:
                shared_k = (S * r - 8) // L
                full0 = shared_k + 1
                shared_mask = hi8
            ones = jnp.minimum(idx_v[pl.ds(L * shared_k, L)], 1) * shared_mask
            for k in range(full0, full0 + 12):
                ones = ones + jnp.minimum(idx_v[pl.ds(L * k, L)], 1)
            cnt = ones[0]
            for j in range(1, L):
                cnt = cnt + ones[j]
            denom = jnp.maximum(
                jnp.broadcast_to(cnt, (L,)).astype(jnp.float32), 1.0
            )
            inv = 1.0 / denom
            for l in range(NVR):
                out_v[r, pl.ds(L * l, L)] = accs[l] * inv

        pltpu.sync_copy(out_v, out_hbm.at[pl.ds(row0, CB)])
        return carry

    lax.fori_loop(0, NCH, chunk_body, 0)


def kernel(x, table):
    return _emb_pool(x.reshape(-1), table)


# speedup vs baseline: 2.7660x; 1.1073x over previous
import functools

import jax
import jax.numpy as jnp
from jax import lax
from jax.experimental import pallas as pl
from jax.experimental.pallas import tpu as pltpu
from jax.experimental.pallas import tpu_sc as plsc

B = 16384
S = 200
D = 64
NC = 2
NS = 16
NW = NC * NS
BPW = B // NW
CB = 4
NCH = BPW // CB
G = 80
NG = (CB * S) // G
L = 16
NVR = D // L

_mesh = plsc.VectorSubcoreMesh(core_axis_name="c", subcore_axis_name="s")


@functools.partial(
    pl.kernel,
    mesh=_mesh,
    out_type=jax.ShapeDtypeStruct((B, D), jnp.float32),
    scratch_types=[
        pltpu.VMEM((CB * S,), jnp.int32),
        pltpu.VMEM((CB * S, D), jnp.float32),
        pltpu.VMEM((2, CB, D), jnp.float32),
        pltpu.SemaphoreType.DMA,
        pltpu.SemaphoreType.DMA((2,)),
    ],
    compiler_params=pltpu.CompilerParams(use_tc_tiling_on_sc=False),
)
def _emb_pool(x_hbm, table_hbm, out_hbm, idx_v, rows_v, out_v, gat_sem, out_sem):
    wid = lax.axis_index("s") * NC + lax.axis_index("c")
    lane = lax.broadcasted_iota(jnp.int32, (L,), 0)
    lo8 = jnp.minimum(jnp.maximum(8 - lane, 0), 1)
    hi8 = 1 - lo8

    def chunk_body(i, carry):
        row0 = wid * BPW + i * CB
        cur = jnp.bitwise_and(i, 1)
        pltpu.sync_copy(x_hbm.at[pl.ds(row0 * S, CB * S)], idx_v)
        for j in range(NG):
            pltpu.async_copy(
                table_hbm.at[idx_v.at[pl.ds(j * G, G)]],
                rows_v.at[pl.ds(j * G, G)],
                gat_sem,
            )

        invs = []
        for r in range(CB):
            if r % 2 == 0:
                full0 = (S * r) // L
                shared_k = full0 + 12
                shared_mask = lo8
            else:
                shared_k = (S * r - 8) // L
                full0 = shared_k + 1
                shared_mask = hi8
            ones = jnp.minimum(idx_v[pl.ds(L * shared_k, L)], 1) * shared_mask
            for k in range(full0, full0 + 12):
                ones = ones + jnp.minimum(idx_v[pl.ds(L * k, L)], 1)
            cnt = ones[0]
            for j in range(1, L):
                cnt = cnt + ones[j]
            denom = jnp.maximum(
                jnp.broadcast_to(cnt, (L,)).astype(jnp.float32), 1.0
            )
            invs.append(1.0 / denom)

        for j in range(NG):
            pltpu.make_async_copy(
                table_hbm.at[idx_v.at[pl.ds(j * G, G)]],
                rows_v.at[pl.ds(j * G, G)],
                gat_sem,
            ).wait()

        @pl.when(i >= 2)
        def _():
            pltpu.make_async_copy(
                out_v.at[cur],
                out_hbm.at[pl.ds(row0 - 2 * CB, CB)],
                out_sem.at[cur],
            ).wait()

        for r in range(CB):
            def srow(s, accs):
                row = r * S + s
                return tuple(
                    a + rows_v[row, pl.ds(L * l, L)] for l, a in enumerate(accs)
                )

            accs = lax.fori_loop(
                0,
                S,
                srow,
                tuple(jnp.zeros((L,), jnp.float32) for _ in range(NVR)),
                unroll=8,
            )
            for l in range(NVR):
                out_v[cur, r, pl.ds(L * l, L)] = accs[l] * invs[r]

        pltpu.async_copy(
            out_v.at[cur], out_hbm.at[pl.ds(row0, CB)], out_sem.at[cur]
        )
        return carry

    lax.fori_loop(0, NCH, chunk_body, 0)

    for t in (NCH - 2, NCH - 1):
        pltpu.make_async_copy(
            out_v.at[t % 2],
            out_hbm.at[pl.ds(wid * BPW + t * CB, CB)],
            out_sem.at[t % 2],
        ).wait()


def kernel(x, table):
    return _emb_pool(x.reshape(-1), table)


# speedup vs baseline: 3.7775x; 1.3657x over previous
import functools

import jax
import jax.numpy as jnp
from jax import lax
from jax.experimental import pallas as pl
from jax.experimental.pallas import tpu as pltpu
from jax.experimental.pallas import tpu_sc as plsc

B = 16384
S = 200
D = 64
NC = 2
NS = 16
NW = NC * NS
BPW = B // NW
CB = 4
NCH = BPW // CB
G = 80
NG = (CB * S) // G
L = 16
NVR = D // L

_mesh = plsc.VectorSubcoreMesh(core_axis_name="c", subcore_axis_name="s")


@functools.partial(
    pl.kernel,
    mesh=_mesh,
    out_type=jax.ShapeDtypeStruct((B, D), jnp.float32),
    scratch_types=[
        pltpu.VMEM((3, CB * S), jnp.int32),
        pltpu.VMEM((2, CB * S, D), jnp.float32),
        pltpu.VMEM((2, CB, D), jnp.float32),
        pltpu.SemaphoreType.DMA((3,)),
        pltpu.SemaphoreType.DMA((2,)),
        pltpu.SemaphoreType.DMA((2,)),
    ],
    compiler_params=pltpu.CompilerParams(use_tc_tiling_on_sc=False),
)
def _emb_pool(
    x_hbm, table_hbm, out_hbm, idx_v, rows_v, out_v, idx_sem, gat_sem, out_sem
):
    wid = lax.axis_index("s") * NC + lax.axis_index("c")
    lane = lax.broadcasted_iota(jnp.int32, (L,), 0)
    lo8 = jnp.minimum(jnp.maximum(8 - lane, 0), 1)
    hi8 = 1 - lo8

    def idx_copy(c, sl):
        return pltpu.make_async_copy(
            x_hbm.at[pl.ds((wid * BPW + c * CB) * S, CB * S)],
            idx_v.at[sl],
            idx_sem.at[sl],
        )

    def fire_gathers(bsl, gsl):
        for j in range(NG):
            pltpu.async_copy(
                table_hbm.at[idx_v.at[bsl, pl.ds(j * G, G)]],
                rows_v.at[gsl, pl.ds(j * G, G)],
                gat_sem.at[gsl],
            )

    def wait_gathers(bsl, gsl):
        for j in range(NG):
            pltpu.make_async_copy(
                table_hbm.at[idx_v.at[bsl, pl.ds(j * G, G)]],
                rows_v.at[gsl, pl.ds(j * G, G)],
                gat_sem.at[gsl],
            ).wait()

    idx_copy(0, 0).start()
    idx_copy(0, 0).wait()
    fire_gathers(0, 0)
    idx_copy(1, 1).start()

    def chunk_body(i, carry):
        row0 = wid * BPW + i * CB
        cur = jnp.bitwise_and(i, 1)
        nxt = jnp.bitwise_and(i + 1, 1)
        bsl = lax.rem(i, 3)
        bsl1 = lax.rem(i + 1, 3)
        bsl2 = lax.rem(i + 2, 3)

        @pl.when(i + 1 < NCH)
        def _():
            idx_copy(i + 1, bsl1).wait()
            fire_gathers(bsl1, nxt)

        @pl.when(i + 2 < NCH)
        def _():
            idx_copy(i + 2, bsl2).start()

        invs = []
        for r in range(CB):
            if r % 2 == 0:
                full0 = (S * r) // L
                shared_k = full0 + 12
                shared_mask = lo8
            else:
                shared_k = (S * r - 8) // L
                full0 = shared_k + 1
                shared_mask = hi8
            ones = (
                jnp.minimum(idx_v[bsl, pl.ds(L * shared_k, L)], 1) * shared_mask
            )
            for k in range(full0, full0 + 12):
                ones = ones + jnp.minimum(idx_v[bsl, pl.ds(L * k, L)], 1)
            cnt = ones[0]
            for j in range(1, L):
                cnt = cnt + ones[j]
            denom = jnp.maximum(
                jnp.broadcast_to(cnt, (L,)).astype(jnp.float32), 1.0
            )
            invs.append(1.0 / denom)

        wait_gathers(bsl, cur)

        @pl.when(i >= 2)
        def _():
            pltpu.make_async_copy(
                out_v.at[cur],
                out_hbm.at[pl.ds(row0 - 2 * CB, CB)],
                out_sem.at[cur],
            ).wait()

        for r in range(CB):
            def srow(s, accs):
                row = r * S + s
                return tuple(
                    a + rows_v[cur, row, pl.ds(L * l, L)]
                    for l, a in enumerate(accs)
                )

            accs = lax.fori_loop(
                0,
                S,
                srow,
                tuple(jnp.zeros((L,), jnp.float32) for _ in range(NVR)),
                unroll=8,
            )
            for l in range(NVR):
                out_v[cur, r, pl.ds(L * l, L)] = accs[l] * invs[r]

        pltpu.async_copy(
            out_v.at[cur], out_hbm.at[pl.ds(row0, CB)], out_sem.at[cur]
        )
        return carry

    lax.fori_loop(0, NCH, chunk_body, 0)

    for t in (NCH - 2, NCH - 1):
        pltpu.make_async_copy(
            out_v.at[t % 2],
            out_hbm.at[pl.ds(wid * BPW + t * CB, CB)],
            out_sem.at[t % 2],
        ).wait()


def kernel(x, table):
    return _emb_pool(x.reshape(-1), table)
